# baseline (speedup 1.0000x reference)
import functools

import jax
import jax.numpy as jnp
from jax import lax
from jax.experimental import pallas as pl
from jax.experimental.pallas import tpu as pltpu
from jax.experimental.pallas import tpu_sc as plsc

_EPS = 1e-14

_NC = 2
_NS = 16
_CH = 128


_NB = 6
_AH = 3


def _make_sc_segsum(N_pad, B, K0, K1, rem):
    mesh = plsc.VectorSubcoreMesh(core_axis_name="c", subcore_axis_name="s")
    rows_per = N_pad // _NS
    Kmax = max(K0, K1)

    def body(hv_hbm, ei_hbm, zero_hbm, out_hbm,
             ei_v, rows_v, acc_sh, hv_sh, gsem, ssem):
        cid = lax.axis_index("c")
        sid = lax.axis_index("s")
        pltpu.sync_copy(zero_hbm, acc_sh.at[pl.ds(sid * rows_per, rows_per)])
        stage_rows = hv_hbm.shape[0] // _NS
        pltpu.sync_copy(hv_hbm.at[pl.ds(sid * stage_rows, stage_rows)],
                        hv_sh.at[pl.ds(sid * stage_rows, stage_rows)])

        def gather(j, b):
            pltpu.async_copy(hv_sh.at[ei_v.at[j, 0]], rows_v.at[b], gsem.at[b])

        def run(Kc, chunk_base, extra):
            pltpu.sync_copy(ei_hbm.at[pl.ds(chunk_base, Kc)],
                            ei_v.at[pl.ds(0, Kc)])
            if extra and rem:
                @pl.when(sid < rem)
                def _():
                    pltpu.sync_copy(ei_hbm.at[pl.ds(_NS * (K0 + K1) + sid, 1)],
                                    ei_v.at[pl.ds(Kc, 1)])
            plsc.subcore_barrier()

            for a in range(_AH):
                gather(a, a)

            def group(g, carry):
                for b in range(_NB):
                    j = g * _NB + b
                    pltpu.make_async_copy(hv_sh.at[ei_v.at[j, 0]],
                                          rows_v.at[b], gsem.at[b]).wait()
                    pltpu.async_copy(rows_v.at[b], acc_sh.at[ei_v.at[j, 1]],
                                     ssem.at[b], add=True)
                    jn = j + _AH
                    bn = (b + _AH) % _NB

                    @pl.when(jn < Kc)
                    def _():
                        @pl.when(jn >= _NB)
                        def _():
                            pltpu.make_async_copy(
                                rows_v.at[bn],
                                acc_sh.at[ei_v.at[jn - _NB, 1]],
                                ssem.at[bn]).wait()
                        gather(jn, bn)
                return carry

            lax.fori_loop(0, Kc // _NB, group, 0)
            for b in range(_NB):
                pltpu.make_async_copy(rows_v.at[b],
                                      acc_sh.at[ei_v.at[Kc - _NB + b, 1]],
                                      ssem.at[b]).wait()
            if extra and rem:
                @pl.when(sid < rem)
                def _():
                    pltpu.async_copy(hv_sh.at[ei_v.at[Kc, 0]], rows_v.at[0],
                                     gsem.at[0]).wait()
                    pltpu.sync_copy(rows_v.at[0], acc_sh.at[ei_v.at[Kc, 1]],
                                    add=True)

        @pl.when(cid == 0)
        def _():
            run(K0, sid * (K0 + K1), True)

        @pl.when(cid == 1)
        def _():
            run(K1, sid * (K0 + K1) + K0, False)

        plsc.subcore_barrier()
        pltpu.sync_copy(acc_sh.at[pl.ds(sid * rows_per, rows_per)],
                        out_hbm.at[cid].at[pl.ds(sid * rows_per, rows_per)])

    return pl.kernel(
        body,
        out_type=jax.ShapeDtypeStruct((_NC, N_pad, B), jnp.float32),
        mesh=mesh,
        scratch_types=[
            pltpu.VMEM((Kmax + 1, 2, _CH), jnp.int32),
            pltpu.VMEM((_NB, _CH, B), jnp.float32),
            pltpu.VMEM_SHARED((N_pad, B), jnp.float32),
            pltpu.VMEM_SHARED((N_pad, B), jnp.float32),
            pltpu.SemaphoreType.DMA((_NB,)),
            pltpu.SemaphoreType.DMA((_NB,)),
        ],
        compiler_params=pltpu.CompilerParams(use_tc_tiling_on_sc=False),
    )


def _clip_norm(v):
    v = jnp.where(v < _EPS, 0.0, v)
    denom = jnp.maximum(_EPS, jnp.sum(v, axis=-1, keepdims=True))
    return v / denom


def _tc_neural_body(he_ref, pe_ref, ee_ref, neural_ref):
    z = he_ref[...] + pe_ref[...]
    score = lax.dot_general(z, ee_ref[...], (((1,), (1,)), ((), ())),
                            preferred_element_type=jnp.float32)
    m = jnp.max(score, axis=-1, keepdims=True)
    p = jnp.exp(score - m)
    neural_ref[...] = p / jnp.sum(p, axis=-1, keepdims=True)


def _tc_combine_body(N, neural_ref, ee_ref, parts_ref, oemb_ref, ovec_ref):
    sym_t = parts_ref[0] + parts_ref[1]
    sym = sym_t.T[:, :N]
    sym = _clip_norm(sym)

    out_vec = _clip_norm(_clip_norm(sym + neural_ref[...]))
    ovec_ref[...] = out_vec
    oemb_ref[...] = lax.dot_general(out_vec, ee_ref[...],
                                    (((1,), (0,)), ((), ())),
                                    preferred_element_type=jnp.float32)


def kernel(head_vector, head_emb, pred_emb, entity_embedding, edge_index):
    B, N = head_vector.shape
    d = head_emb.shape[1]
    E = edge_index.shape[1]
    n_chunks = E // _CH
    KT = n_chunks // _NS
    rem = n_chunks - KT * _NS
    K0 = (int(round(KT * 0.5)) // _NB) * _NB
    K1 = KT - K0
    N_pad = -(-N // 128) * 128

    hv_t = head_vector.T
    ei = edge_index.reshape(2, n_chunks, _CH).transpose(1, 0, 2)
    zeros = jnp.zeros((N_pad // _NS, B), jnp.float32)

    parts = _make_sc_segsum(N_pad, B, K0, K1, rem)(hv_t, ei, zeros)

    neural = pl.pallas_call(
        _tc_neural_body,
        out_shape=jax.ShapeDtypeStruct((B, N), jnp.float32),
    )(head_emb, pred_emb, entity_embedding)
    oemb, ovec = pl.pallas_call(
        functools.partial(_tc_combine_body, N),
        out_shape=(jax.ShapeDtypeStruct((B, d), jnp.float32),
                   jax.ShapeDtypeStruct((B, N), jnp.float32)),
    )(neural, entity_embedding, parts)
    return (oemb, ovec)


# speedup vs baseline: 1.0102x; 1.0102x over previous
import functools

import jax
import jax.numpy as jnp
from jax import lax
from jax.experimental import pallas as pl
from jax.experimental.pallas import tpu as pltpu
from jax.experimental.pallas import tpu_sc as plsc

_EPS = 1e-14

_NC = 2
_NS = 16
_CH = 128


_NB = 6
_AH = 3


def _make_sc_segsum(N_pad, B, K0, K1, rem):
    mesh = plsc.VectorSubcoreMesh(core_axis_name="c", subcore_axis_name="s")
    rows_per = N_pad // _NS
    Kmax = max(K0, K1)

    def body(hv_hbm, ei_hbm, zero_hbm, out_hbm,
             ei_v, rows_v, acc_sh, hv_sh, gsem, ssem):
        cid = lax.axis_index("c")
        sid = lax.axis_index("s")
        pltpu.sync_copy(zero_hbm, acc_sh.at[pl.ds(sid * rows_per, rows_per)])
        stage_rows = hv_hbm.shape[0] // _NS
        pltpu.sync_copy(hv_hbm.at[pl.ds(sid * stage_rows, stage_rows)],
                        hv_sh.at[pl.ds(sid * stage_rows, stage_rows)])

        def gather(j, b):
            pltpu.async_copy(hv_sh.at[ei_v.at[j, 0]], rows_v.at[b], gsem.at[b])

        def run(Kc, chunk_base, extra):
            pltpu.sync_copy(ei_hbm.at[pl.ds(chunk_base, Kc)],
                            ei_v.at[pl.ds(0, Kc)])
            if extra and rem:
                @pl.when(sid < rem)
                def _():
                    pltpu.sync_copy(ei_hbm.at[pl.ds(_NS * (K0 + K1) + sid, 1)],
                                    ei_v.at[pl.ds(Kc, 1)])
            plsc.subcore_barrier()

            for a in range(_AH):
                gather(a, a)

            def group(g, carry):
                for b in range(_NB):
                    j = g * _NB + b
                    pltpu.make_async_copy(hv_sh.at[ei_v.at[j, 0]],
                                          rows_v.at[b], gsem.at[b]).wait()
                    pltpu.async_copy(rows_v.at[b], acc_sh.at[ei_v.at[j, 1]],
                                     ssem.at[b], add=True)
                    jn = j + _AH
                    bn = (b + _AH) % _NB

                    @pl.when(jn < Kc)
                    def _():
                        @pl.when(jn >= _NB)
                        def _():
                            pltpu.make_async_copy(
                                rows_v.at[bn],
                                acc_sh.at[ei_v.at[jn - _NB, 1]],
                                ssem.at[bn]).wait()
                        gather(jn, bn)
                return carry

            lax.fori_loop(0, Kc // _NB, group, 0)
            for b in range(_NB):
                pltpu.make_async_copy(rows_v.at[b],
                                      acc_sh.at[ei_v.at[Kc - _NB + b, 1]],
                                      ssem.at[b]).wait()
            if extra and rem:
                @pl.when(sid < rem)
                def _():
                    pltpu.async_copy(hv_sh.at[ei_v.at[Kc, 0]], rows_v.at[0],
                                     gsem.at[0]).wait()
                    pltpu.sync_copy(rows_v.at[0], acc_sh.at[ei_v.at[Kc, 1]],
                                    add=True)

        @pl.when(cid == 0)
        def _():
            run(K0, sid * (K0 + K1), True)

        @pl.when(cid == 1)
        def _():
            run(K1, sid * (K0 + K1) + K0, False)

        plsc.subcore_barrier()
        pltpu.sync_copy(acc_sh.at[pl.ds(sid * rows_per, rows_per)],
                        out_hbm.at[cid].at[pl.ds(sid * rows_per, rows_per)])

    return pl.kernel(
        body,
        out_type=jax.ShapeDtypeStruct((_NC, N_pad, B), jnp.float32),
        mesh=mesh,
        scratch_types=[
            pltpu.VMEM((Kmax + 1, 2, _CH), jnp.int32),
            pltpu.VMEM((_NB, _CH, B), jnp.float32),
            pltpu.VMEM_SHARED((N_pad, B), jnp.float32),
            pltpu.VMEM_SHARED((N_pad, B), jnp.float32),
            pltpu.SemaphoreType.DMA((_NB,)),
            pltpu.SemaphoreType.DMA((_NB,)),
        ],
        compiler_params=pltpu.CompilerParams(use_tc_tiling_on_sc=False),
    )


def _clip_norm(v):
    v = jnp.where(v < _EPS, 0.0, v)
    denom = jnp.maximum(_EPS, jnp.sum(v, axis=-1, keepdims=True))
    return v / denom


def _tc_body(N, he_ref, pe_ref, ee_ref, parts_ref, oemb_ref, ovec_ref):
    z = he_ref[...] + pe_ref[...]
    ee = ee_ref[...]
    score = lax.dot_general(z, ee, (((1,), (1,)), ((), ())),
                            preferred_element_type=jnp.float32)
    m = jnp.max(score, axis=-1, keepdims=True)
    p = jnp.exp(score - m)
    neural = p / jnp.sum(p, axis=-1, keepdims=True)

    sym_t = parts_ref[0] + parts_ref[1]
    sym = sym_t.T[:, :N]
    sym = _clip_norm(sym)

    out_vec = _clip_norm(_clip_norm(sym + neural))
    ovec_ref[...] = out_vec
    oemb_ref[...] = lax.dot_general(out_vec, ee, (((1,), (0,)), ((), ())),
                                    preferred_element_type=jnp.float32)


def kernel(head_vector, head_emb, pred_emb, entity_embedding, edge_index):
    B, N = head_vector.shape
    d = head_emb.shape[1]
    E = edge_index.shape[1]
    n_chunks = E // _CH
    KT = n_chunks // _NS
    rem = n_chunks - KT * _NS
    K0 = (int(round(KT * 0.5)) // _NB) * _NB
    K1 = KT - K0
    N_pad = -(-N // 128) * 128

    hv_t = head_vector.T
    ei = edge_index.reshape(2, n_chunks, _CH).transpose(1, 0, 2)
    zeros = jnp.zeros((N_pad // _NS, B), jnp.float32)

    parts = _make_sc_segsum(N_pad, B, K0, K1, rem)(hv_t, ei, zeros)

    oemb, ovec = pl.pallas_call(
        functools.partial(_tc_body, N),
        out_shape=(jax.ShapeDtypeStruct((B, d), jnp.float32),
                   jax.ShapeDtypeStruct((B, N), jnp.float32)),
    )(head_emb, pred_emb, entity_embedding, parts)
    return (oemb, ovec)


# speedup vs baseline: 1.0119x; 1.0017x over previous
import functools

import jax
import jax.numpy as jnp
from jax import lax
from jax.experimental import pallas as pl
from jax.experimental.pallas import tpu as pltpu
from jax.experimental.pallas import tpu_sc as plsc

_EPS = 1e-14

_NC = 2
_NS = 16
_CH = 128


_NB = 6
_AH = 4


def _make_sc_segsum(N_pad, B, K0, K1, rem):
    mesh = plsc.VectorSubcoreMesh(core_axis_name="c", subcore_axis_name="s")
    rows_per = N_pad // _NS
    Kmax = max(K0, K1)

    def body(hv_hbm, ei_hbm, zero_hbm, out_hbm,
             ei_v, rows_v, acc_sh, hv_sh, gsem, ssem):
        cid = lax.axis_index("c")
        sid = lax.axis_index("s")
        pltpu.sync_copy(zero_hbm, acc_sh.at[pl.ds(sid * rows_per, rows_per)])
        stage_rows = hv_hbm.shape[0] // _NS
        pltpu.sync_copy(hv_hbm.at[pl.ds(sid * stage_rows, stage_rows)],
                        hv_sh.at[pl.ds(sid * stage_rows, stage_rows)])

        def gather(j, b):
            pltpu.async_copy(hv_sh.at[ei_v.at[j, 0]], rows_v.at[b], gsem.at[b])

        def run(Kc, chunk_base, extra):
            pltpu.sync_copy(ei_hbm.at[pl.ds(chunk_base, Kc)],
                            ei_v.at[pl.ds(0, Kc)])
            if extra and rem:
                @pl.when(sid < rem)
                def _():
                    pltpu.sync_copy(ei_hbm.at[pl.ds(_NS * (K0 + K1) + sid, 1)],
                                    ei_v.at[pl.ds(Kc, 1)])
            plsc.subcore_barrier()

            for a in range(_AH):
                gather(a, a)

            def group(g, carry):
                for b in range(_NB):
                    j = g * _NB + b
                    pltpu.make_async_copy(hv_sh.at[ei_v.at[j, 0]],
                                          rows_v.at[b], gsem.at[b]).wait()
                    pltpu.async_copy(rows_v.at[b], acc_sh.at[ei_v.at[j, 1]],
                                     ssem.at[b], add=True)
                    jn = j + _AH
                    bn = (b + _AH) % _NB

                    @pl.when(jn < Kc)
                    def _():
                        @pl.when(jn >= _NB)
                        def _():
                            pltpu.make_async_copy(
                                rows_v.at[bn],
                                acc_sh.at[ei_v.at[jn - _NB, 1]],
                                ssem.at[bn]).wait()
                        gather(jn, bn)
                return carry

            lax.fori_loop(0, Kc // _NB, group, 0)
            for b in range(_NB):
                pltpu.make_async_copy(rows_v.at[b],
                                      acc_sh.at[ei_v.at[Kc - _NB + b, 1]],
                                      ssem.at[b]).wait()
            if extra and rem:
                @pl.when(sid < rem)
                def _():
                    pltpu.async_copy(hv_sh.at[ei_v.at[Kc, 0]], rows_v.at[0],
                                     gsem.at[0]).wait()
                    pltpu.sync_copy(rows_v.at[0], acc_sh.at[ei_v.at[Kc, 1]],
                                    add=True)

        @pl.when(cid == 0)
        def _():
            run(K0, sid * (K0 + K1), True)

        @pl.when(cid == 1)
        def _():
            run(K1, sid * (K0 + K1) + K0, False)

        plsc.subcore_barrier()
        pltpu.sync_copy(acc_sh.at[pl.ds(sid * rows_per, rows_per)],
                        out_hbm.at[cid].at[pl.ds(sid * rows_per, rows_per)])

    return pl.kernel(
        body,
        out_type=jax.ShapeDtypeStruct((_NC, N_pad, B), jnp.float32),
        mesh=mesh,
        scratch_types=[
            pltpu.VMEM((Kmax + 1, 2, _CH), jnp.int32),
            pltpu.VMEM((_NB, _CH, B), jnp.float32),
            pltpu.VMEM_SHARED((N_pad, B), jnp.float32),
            pltpu.VMEM_SHARED((N_pad, B), jnp.float32),
            pltpu.SemaphoreType.DMA((_NB,)),
            pltpu.SemaphoreType.DMA((_NB,)),
        ],
        compiler_params=pltpu.CompilerParams(use_tc_tiling_on_sc=False),
    )


def _clip_norm(v):
    v = jnp.where(v < _EPS, 0.0, v)
    denom = jnp.maximum(_EPS, jnp.sum(v, axis=-1, keepdims=True))
    return v / denom


def _tc_body(N, he_ref, pe_ref, ee_ref, parts_ref, oemb_ref, ovec_ref):
    z = he_ref[...] + pe_ref[...]
    ee = ee_ref[...]
    score = lax.dot_general(z, ee, (((1,), (1,)), ((), ())),
                            preferred_element_type=jnp.float32)
    m = jnp.max(score, axis=-1, keepdims=True)
    p = jnp.exp(score - m)
    neural = p / jnp.sum(p, axis=-1, keepdims=True)

    sym_t = parts_ref[0] + parts_ref[1]
    sym = sym_t.T[:, :N]
    sym = _clip_norm(sym)

    out_vec = _clip_norm(_clip_norm(sym + neural))
    ovec_ref[...] = out_vec
    oemb_ref[...] = lax.dot_general(out_vec, ee, (((1,), (0,)), ((), ())),
                                    preferred_element_type=jnp.float32)


def kernel(head_vector, head_emb, pred_emb, entity_embedding, edge_index):
    B, N = head_vector.shape
    d = head_emb.shape[1]
    E = edge_index.shape[1]
    n_chunks = E // _CH
    KT = n_chunks // _NS
    rem = n_chunks - KT * _NS
    K0 = (int(round(KT * 0.5)) // _NB) * _NB
    K1 = KT - K0
    N_pad = -(-N // 128) * 128

    hv_t = head_vector.T
    ei = edge_index.reshape(2, n_chunks, _CH).transpose(1, 0, 2)
    zeros = jnp.zeros((N_pad // _NS, B), jnp.float32)

    parts = _make_sc_segsum(N_pad, B, K0, K1, rem)(hv_t, ei, zeros)

    oemb, ovec = pl.pallas_call(
        functools.partial(_tc_body, N),
        out_shape=(jax.ShapeDtypeStruct((B, d), jnp.float32),
                   jax.ShapeDtypeStruct((B, N), jnp.float32)),
    )(head_emb, pred_emb, entity_embedding, parts)
    return (oemb, ovec)
